# baseline (speedup 1.0000x reference)
import functools

import jax
import jax.numpy as jnp
from jax import lax
from jax.experimental import pallas as pl
from jax.experimental.pallas import tpu as pltpu
from jax.experimental.pallas import tpu_sc as plsc

DIM = 64
CHUNK = 128
NBUF = 5


@functools.lru_cache(maxsize=None)
def _make_kernel(total_rows, dim):
    info = plsc.get_sparse_core_info()
    nc, ns = info.num_cores, info.num_subcores
    nw = nc * ns
    rows_per_w = total_rows // nw
    n_chunks = rows_per_w // CHUNK
    n_groups = n_chunks // NBUF
    assert rows_per_w * nw == total_rows
    assert n_chunks * CHUNK == rows_per_w
    assert n_groups * NBUF == n_chunks

    mesh = plsc.VectorSubcoreMesh(core_axis_name="c", subcore_axis_name="s")
    scratch = [pltpu.VMEM((rows_per_w,), jnp.int32)]
    scratch += [pltpu.VMEM((CHUNK, dim), jnp.float32) for _ in range(NBUF)]
    scratch += [pltpu.SemaphoreType.DMA for _ in range(NBUF)]

    @functools.partial(
        pl.kernel,
        mesh=mesh,
        out_type=jax.ShapeDtypeStruct((total_rows, dim), jnp.float32),
        scratch_types=scratch,
        compiler_params=pltpu.CompilerParams(use_tc_tiling_on_sc=False),
    )
    def gather_kernel(idx_hbm, table_hbm, out_hbm, idx_v, *rest):
        bufs = rest[:NBUF]
        sems = rest[NBUF:]
        wid = lax.axis_index("s") * nc + lax.axis_index("c")

        base = pl.multiple_of(wid * rows_per_w, rows_per_w)
        pltpu.sync_copy(idx_hbm.at[pl.ds(base, rows_per_w)], idx_v)

        def fire(j, b):
            start = pl.multiple_of(j * CHUNK, CHUNK)
            pltpu.make_async_copy(
                table_hbm.at[idx_v.at[pl.ds(start, CHUNK)]], bufs[b], sems[b]
            ).start()

        def drain(b):
            pltpu.make_async_copy(
                table_hbm.at[idx_v.at[pl.ds(0, CHUNK)]], bufs[b], sems[b]
            ).wait()

        def store(j, b):
            start = pl.multiple_of(base + j * CHUNK, CHUNK)
            pltpu.sync_copy(bufs[b], out_hbm.at[pl.ds(start, CHUNK)])

        for b in range(NBUF):
            fire(b, b)

        def body(g, carry):
            for b in range(NBUF):
                j = g * NBUF + b
                drain(b)
                store(j, b)
                fire(j + NBUF, b)
            return carry

        lax.fori_loop(0, n_groups - 1, body, 0)

        for b in range(NBUF):
            drain(b)
            store((n_groups - 1) * NBUF + b, b)

    return gather_kernel


def kernel(token_ids, table):
    batch, hist = token_ids.shape
    total = batch * hist
    dim = table.shape[1]
    idx = token_ids.reshape(total).astype(jnp.int32)
    out = _make_kernel(total, dim)(idx, table)
    return out.reshape(batch, hist, dim)


# speedup vs baseline: 1.0026x; 1.0026x over previous
import functools

import jax
import jax.numpy as jnp
from jax import lax
from jax.experimental import pallas as pl
from jax.experimental.pallas import tpu as pltpu
from jax.experimental.pallas import tpu_sc as plsc

PDIM = 128
CHUNK = 128
NBUF = 5


@functools.lru_cache(maxsize=None)
def _make_kernel(total_rows):
    info = plsc.get_sparse_core_info()
    nc, ns = info.num_cores, info.num_subcores
    nw = nc * ns
    rows_per_w = total_rows // nw
    n_chunks = rows_per_w // CHUNK
    n_groups = n_chunks // NBUF
    assert rows_per_w * nw == total_rows
    assert n_chunks * CHUNK == rows_per_w
    assert n_groups * NBUF == n_chunks

    mesh = plsc.VectorSubcoreMesh(core_axis_name="c", subcore_axis_name="s")
    scratch = [pltpu.VMEM((rows_per_w,), jnp.int32)]
    scratch += [pltpu.VMEM((CHUNK, PDIM), jnp.float32) for _ in range(NBUF)]
    scratch += [pltpu.SemaphoreType.DMA for _ in range(NBUF)]

    @functools.partial(
        pl.kernel,
        mesh=mesh,
        out_type=jax.ShapeDtypeStruct((total_rows, PDIM), jnp.float32),
        scratch_types=scratch,
    )
    def gather_kernel(idx_hbm, table_hbm, out_hbm, idx_v, *rest):
        bufs = rest[:NBUF]
        sems = rest[NBUF:]
        wid = lax.axis_index("s") * nc + lax.axis_index("c")

        base = pl.multiple_of(wid * rows_per_w, rows_per_w)
        pltpu.sync_copy(idx_hbm.at[pl.ds(base, rows_per_w)], idx_v)

        def fire(j, b):
            start = pl.multiple_of(j * CHUNK, CHUNK)
            pltpu.make_async_copy(
                table_hbm.at[idx_v.at[pl.ds(start, CHUNK)]], bufs[b], sems[b]
            ).start()

        def drain(b):
            pltpu.make_async_copy(
                table_hbm.at[idx_v.at[pl.ds(0, CHUNK)]], bufs[b], sems[b]
            ).wait()

        def store(j, b):
            start = pl.multiple_of(base + j * CHUNK, CHUNK)
            pltpu.sync_copy(bufs[b], out_hbm.at[pl.ds(start, CHUNK)])

        for b in range(NBUF):
            fire(b, b)

        def body(g, carry):
            for b in range(NBUF):
                j = g * NBUF + b
                drain(b)
                store(j, b)
                fire(j + NBUF, b)
            return carry

        lax.fori_loop(0, n_groups - 1, body, 0)

        for b in range(NBUF):
            drain(b)
            store((n_groups - 1) * NBUF + b, b)

    return gather_kernel


def kernel(token_ids, table):
    batch, hist = token_ids.shape
    total = batch * hist
    dim = table.shape[1]
    idx = token_ids.reshape(total).astype(jnp.int32)
    table_p = jnp.pad(table, ((0, 0), (0, PDIM - dim)))
    out = _make_kernel(total)(idx, table_p)
    return out[:, :dim].reshape(batch, hist, dim)
